# baseline (speedup 1.0000x reference)
import functools

import jax
import jax.numpy as jnp
from jax import lax
from jax.experimental import pallas as pl
from jax.experimental.pallas import tpu as pltpu
from jax.experimental.pallas import tpu_sc as plsc

B = 16384
D = 64
DV, DA, DT = 512, 128, 768

NC, NS = 2, 16
NW = NC * NS
H = B // 2
BPW = H // NW
MCHUNK = 64
UCHUNK = 128

_sc_mesh = plsc.VectorSubcoreMesh(core_axis_name="c", subcore_axis_name="s")


def _mgather_body(mid_hbm, vf_hbm, af_hbm, tf_hbm,
                  vout, aout, tout,
                  midx, vbuf, abuf, tbuf, sem):
    wid = lax.axis_index("s") * NC + lax.axis_index("c")
    base = wid * BPW
    pltpu.sync_copy(mid_hbm.at[pl.ds(base, BPW)], midx)
    for c in range(BPW // MCHUNK):
        off = c * MCHUNK
        cv = pltpu.async_copy(vf_hbm.at[midx.at[pl.ds(off, MCHUNK)]], vbuf, sem)
        ca = pltpu.async_copy(af_hbm.at[midx.at[pl.ds(off, MCHUNK)]], abuf, sem)
        ct = pltpu.async_copy(tf_hbm.at[midx.at[pl.ds(off, MCHUNK)]], tbuf, sem)
        cv.wait()
        ca.wait()
        ct.wait()
        pltpu.sync_copy(vbuf, vout.at[pl.ds(base + off, MCHUNK)])
        pltpu.sync_copy(abuf, aout.at[pl.ds(base + off, MCHUNK)])
        pltpu.sync_copy(tbuf, tout.at[pl.ds(base + off, MCHUNK)])


_mgather = pl.kernel(
    _mgather_body,
    out_type=[
        jax.ShapeDtypeStruct((H, DV), jnp.float32),
        jax.ShapeDtypeStruct((H, DA), jnp.float32),
        jax.ShapeDtypeStruct((H, DT), jnp.float32),
    ],
    mesh=_sc_mesh,
    scratch_types=[
        pltpu.VMEM((BPW,), jnp.int32),
        pltpu.VMEM((MCHUNK, DV), jnp.float32),
        pltpu.VMEM((MCHUNK, DA), jnp.float32),
        pltpu.VMEM((MCHUNK, DT), jnp.float32),
        pltpu.SemaphoreType.DMA,
    ],
)


def _ugather_body(uid_hbm, ut_hbm, uout, uidx, ubuf, sem):
    wid = lax.axis_index("s") * NC + lax.axis_index("c")
    base = wid * BPW
    pltpu.sync_copy(uid_hbm.at[pl.ds(base, BPW)], uidx)
    for c in range(BPW // UCHUNK):
        off = c * UCHUNK
        cu = pltpu.async_copy(ut_hbm.at[uidx.at[pl.ds(off, UCHUNK)]], ubuf, sem)
        cu.wait()
        pltpu.sync_copy(ubuf, uout.at[pl.ds(base + off, UCHUNK)])


_ugather = pl.kernel(
    _ugather_body,
    out_type=jax.ShapeDtypeStruct((H, 2 * D), jnp.float32),
    mesh=_sc_mesh,
    scratch_types=[
        pltpu.VMEM((BPW,), jnp.int32),
        pltpu.VMEM((UCHUNK, 2 * D), jnp.float32),
        pltpu.SemaphoreType.DMA,
    ],
)


BT = 512


def _fuse_body(u_ref, v_ref, a_ref, t_ref, wv_ref, wa_ref, wt_ref,
               bmm_ref, wout_ref, bout_ref, o_ref):
    memb = jnp.dot(v_ref[...], wv_ref[...], preferred_element_type=jnp.float32)
    memb += jnp.dot(a_ref[...], wa_ref[...], preferred_element_type=jnp.float32)
    memb += jnp.dot(t_ref[...], wt_ref[...], preferred_element_type=jnp.float32)
    memb += bmm_ref[...]
    mu = jnp.sum(memb * u_ref[:, :D], axis=1, keepdims=True)
    o_ref[...] = jax.nn.sigmoid(mu * wout_ref[0, 0] + bout_ref[0, 0])


def _fuse(uemb, mv, ma, mt, Wv, Wa, Wt, bmm, wout, bout):
    return pl.pallas_call(
        _fuse_body,
        grid=(H // BT,),
        in_specs=[
            pl.BlockSpec((BT, 2 * D), lambda i: (i, 0)),
            pl.BlockSpec((BT, DV), lambda i: (i, 0)),
            pl.BlockSpec((BT, DA), lambda i: (i, 0)),
            pl.BlockSpec((BT, DT), lambda i: (i, 0)),
            pl.BlockSpec((DV, D), lambda i: (0, 0)),
            pl.BlockSpec((DA, D), lambda i: (0, 0)),
            pl.BlockSpec((DT, D), lambda i: (0, 0)),
            pl.BlockSpec((1, D), lambda i: (0, 0)),
            pl.BlockSpec((1, 1), lambda i: (0, 0)),
            pl.BlockSpec((1, 1), lambda i: (0, 0)),
        ],
        out_specs=pl.BlockSpec((BT, 1), lambda i: (i, 0)),
        out_shape=jax.ShapeDtypeStruct((H, 1), jnp.float32),
    )(uemb, mv, ma, mt, Wv, Wa, Wt, bmm, wout, bout)


def kernel(x, user_table, video_feat, audio_feat, text_feat, W_mm, b_mm, W_out, b_out):
    uid = x[0].astype(jnp.int32)
    mid = x[1].astype(jnp.int32)
    ut_pad = jnp.pad(user_table, ((0, 0), (0, D)))
    Wv = W_mm[:DV]
    Wa = W_mm[DV:DV + DA]
    Wt = W_mm[DV + DA:]
    bmm = b_mm.reshape(1, D)
    bout = b_out.reshape(1, 1)

    mv0, ma0, mt0 = _mgather(mid[:H], video_feat, audio_feat, text_feat)
    ue0 = _ugather(uid[:H], ut_pad)
    mv1, ma1, mt1 = _mgather(mid[H:], video_feat, audio_feat, text_feat)
    ue1 = _ugather(uid[H:], ut_pad)

    o0 = _fuse(ue0, mv0, ma0, mt0, Wv, Wa, Wt, bmm, W_out, bout)
    o1 = _fuse(ue1, mv1, ma1, mt1, Wv, Wa, Wt, bmm, W_out, bout)
    return jnp.concatenate([o0, o1], axis=0)
